# Initial kernel scaffold; baseline (speedup 1.0000x reference)
import jax
import jax.numpy as jnp
from jax.experimental import pallas as pl


def kernel(input, weights, offset):
    raise NotImplementedError("write your pallas kernel here")


# speedup vs baseline: 2.1267x; 2.1267x over previous
import jax
import jax.numpy as jnp
from jax.experimental import pallas as pl
from jax.experimental.pallas import tpu as pltpu

_BS = 256


def _body(off_ref, w_ref, o_ref):
    del off_ref
    o_ref[...] = jnp.broadcast_to(w_ref[...][None, :, :], o_ref.shape)


def kernel(input, weights, offset):
    bsz, seq_len = input.shape
    num_emb, dim = weights.shape
    nblk = seq_len // _BS
    off = jnp.asarray(offset, jnp.int32).reshape((1,))

    grid_spec = pltpu.PrefetchScalarGridSpec(
        num_scalar_prefetch=1,
        grid=(nblk,),
        in_specs=[
            pl.BlockSpec((_BS, dim), lambda i, off: ((i * _BS + off[0]) // _BS, 0)),
        ],
        out_specs=pl.BlockSpec((bsz, _BS, dim), lambda i, off: (0, i, 0)),
    )
    return pl.pallas_call(
        _body,
        grid_spec=grid_spec,
        out_shape=jax.ShapeDtypeStruct((bsz, seq_len, dim), weights.dtype),
        compiler_params=pltpu.CompilerParams(
            dimension_semantics=("arbitrary",),
        ),
    )(off, weights)
